# baseline (speedup 1.0000x reference)
import functools

import jax
import jax.numpy as jnp
from jax import lax
from jax.experimental import pallas as pl
from jax.experimental.pallas import tpu as pltpu
from jax.experimental.pallas import tpu_sc as plsc

OUT_DIM = 16


def _mlp_table_body(embed_ref, W1_ref, b1_ref, alpha_ref, W2_ref, b2_ref,
                    out_ref):
    e = embed_ref[...]
    h = lax.dot(e, W1_ref[...], preferred_element_type=jnp.float32)
    h = h + b1_ref[...]
    a = alpha_ref[0, 0]
    h = jnp.maximum(h, 0.0) + a * jnp.minimum(h, 0.0)
    out_ref[...] = (lax.dot(h, W2_ref[...], preferred_element_type=jnp.float32)
                    + b2_ref[...])


def _mlp_table(embed, W1, b1, alpha, W2, b2):
    vocab = embed.shape[0]
    return pl.pallas_call(
        _mlp_table_body,
        out_shape=jax.ShapeDtypeStruct((vocab, OUT_DIM), jnp.float32),
        in_specs=[
            pl.BlockSpec(memory_space=pltpu.VMEM),
            pl.BlockSpec(memory_space=pltpu.VMEM),
            pl.BlockSpec(memory_space=pltpu.VMEM),
            pl.BlockSpec(memory_space=pltpu.SMEM),
            pl.BlockSpec(memory_space=pltpu.VMEM),
            pl.BlockSpec(memory_space=pltpu.VMEM),
        ],
        out_specs=pl.BlockSpec(memory_space=pltpu.VMEM),
    )(embed, W1, b1.reshape(1, -1), alpha.reshape(1, 1), W2,
      b2.reshape(1, -1))


def _sc_gather(table_flat, idx2d, nc, ns, nchunks, C):
    nw, per_w = idx2d.shape
    tlen = table_flat.shape[0]
    mesh = plsc.VectorSubcoreMesh(core_axis_name="c", subcore_axis_name="s")
    groups = C // 16
    out_bytes = C * OUT_DIM * 4

    @functools.partial(
        pl.kernel,
        out_type=jax.ShapeDtypeStruct((nw, nchunks, C * OUT_DIM), jnp.float32),
        mesh=mesh,
        scratch_types=[
            pltpu.VMEM((tlen,), jnp.float32),
            pltpu.VMEM((per_w,), jnp.int32),
            pltpu.VMEM((C * OUT_DIM,), jnp.float32),
            pltpu.VMEM((C * OUT_DIM,), jnp.float32),
            pltpu.SemaphoreType.DMA,
            pltpu.SemaphoreType.DMA,
        ],
        compiler_params=pltpu.CompilerParams(use_tc_tiling_on_sc=False,
                                             needs_layout_passes=False),
    )
    def gather_kernel(table_hbm, idx_hbm, out_hbm, table_v, idx_v,
                      rows0, rows1, osem0, osem1):
        wid = lax.axis_index("s") * nc + lax.axis_index("c")
        pltpu.sync_copy(table_hbm, table_v)
        pltpu.sync_copy(idx_hbm.at[wid], idx_v)
        l16 = lax.iota(jnp.int32, 16) * OUT_DIM

        def do_chunk(j, m, rows_buf, osem):
            @pl.when(m >= 1)
            def _wait_prev():
                pltpu.make_async_copy(rows_buf, out_hbm.at[wid, 0],
                                      osem).wait()

            def group(g, carry):
                v = idx_v[pl.ds(j * C + g * 16, 16)]
                a = v * OUT_DIM
                base = g * (16 * OUT_DIM)
                for d in range(OUT_DIM):
                    col = plsc.load_gather(table_v, [a + d])
                    plsc.store_scatter(rows_buf, [l16 + (base + d)], col)
                return carry

            lax.fori_loop(0, groups, group, 0, unroll=2)
            pltpu.async_copy(rows_buf, out_hbm.at[wid, j], osem)

        def body(m, carry):
            do_chunk(2 * m, m, rows0, osem0)
            do_chunk(2 * m + 1, m, rows1, osem1)
            return carry

        lax.fori_loop(0, nchunks // 2, body, 0)
        pltpu.make_async_copy(rows0, out_hbm.at[wid, 0], osem0).wait()
        pltpu.make_async_copy(rows1, out_hbm.at[wid, 0], osem1).wait()

    return gather_kernel(table_flat, idx2d)


def kernel(x, embed, W1, b1, alpha, W2, b2):
    B = x.size
    info = plsc.get_sparse_core_info()
    nc, ns = info.num_cores, info.num_subcores
    nw = nc * ns
    C = 2560
    per_w = B // nw
    nchunks = per_w // C
    assert B == nw * nchunks * C and nchunks % 2 == 0, (B, nw, nchunks, C)

    table = _mlp_table(embed, W1, b1, alpha, W2, b2)
    idx2d = x.reshape(nw, per_w)
    out = _sc_gather(table.reshape(-1), idx2d, nc, ns, nchunks, C)
    return out.reshape(B, OUT_DIM)


# speedup vs baseline: 1.2428x; 1.2428x over previous
import functools

import jax
import jax.numpy as jnp
from jax import lax
from jax.experimental import pallas as pl
from jax.experimental.pallas import tpu as pltpu
from jax.experimental.pallas import tpu_sc as plsc

OUT_DIM = 16


def _mlp_table_body(embed_ref, W1_ref, b1_ref, alpha_ref, W2_ref, b2_ref,
                    out_ref):
    e = embed_ref[...]
    h = lax.dot(e, W1_ref[...], preferred_element_type=jnp.float32)
    h = h + b1_ref[...]
    a = alpha_ref[0, 0]
    h = jnp.maximum(h, 0.0) + a * jnp.minimum(h, 0.0)
    out_ref[...] = (lax.dot(h, W2_ref[...], preferred_element_type=jnp.float32)
                    + b2_ref[...])


def _mlp_table(embed, W1, b1, alpha, W2, b2):
    vocab = embed.shape[0]
    return pl.pallas_call(
        _mlp_table_body,
        out_shape=jax.ShapeDtypeStruct((vocab, OUT_DIM), jnp.float32),
        in_specs=[
            pl.BlockSpec(memory_space=pltpu.VMEM),
            pl.BlockSpec(memory_space=pltpu.VMEM),
            pl.BlockSpec(memory_space=pltpu.VMEM),
            pl.BlockSpec(memory_space=pltpu.SMEM),
            pl.BlockSpec(memory_space=pltpu.VMEM),
            pl.BlockSpec(memory_space=pltpu.VMEM),
        ],
        out_specs=pl.BlockSpec(memory_space=pltpu.VMEM),
    )(embed, W1, b1.reshape(1, -1), alpha.reshape(1, 1), W2,
      b2.reshape(1, -1))


def _sc_gather(table_flat, idx2d, nc, ns, nchunks, C):
    nw, per_w = idx2d.shape
    tlen = table_flat.shape[0]
    mesh = plsc.VectorSubcoreMesh(core_axis_name="c", subcore_axis_name="s")
    groups = C // 16
    out_bytes = C * OUT_DIM * 4

    @functools.partial(
        pl.kernel,
        out_type=jax.ShapeDtypeStruct((nw, nchunks, C * OUT_DIM), jnp.float32),
        mesh=mesh,
        scratch_types=[
            pltpu.VMEM((tlen,), jnp.float32),
            pltpu.VMEM((per_w,), jnp.int32),
            pltpu.VMEM((C * OUT_DIM,), jnp.float32),
            pltpu.VMEM((C * OUT_DIM,), jnp.float32),
            pltpu.SemaphoreType.DMA,
            pltpu.SemaphoreType.DMA,
        ],
        compiler_params=pltpu.CompilerParams(use_tc_tiling_on_sc=False,
                                             needs_layout_passes=False),
    )
    def gather_kernel(table_hbm, idx_hbm, out_hbm, table_v, idx_v,
                      rows0, rows1, osem0, osem1):
        wid = lax.axis_index("s") * nc + lax.axis_index("c")
        pltpu.sync_copy(table_hbm, table_v)
        pltpu.sync_copy(idx_hbm.at[wid], idx_v)
        iota16 = lax.iota(jnp.int32, 16)
        lane_sel = [jnp.full((16, 1), k, jnp.int32) for k in range(16)]
        dnums = lax.GatherDimensionNumbers(offset_dims=(),
                                           collapsed_slice_dims=(0,),
                                           start_index_map=(0,))

        def bcast_lane(vec, k):
            return lax.gather(vec, lane_sel[k], dnums, slice_sizes=(1,),
                              mode=lax.GatherScatterMode.PROMISE_IN_BOUNDS)

        def do_chunk(j, m, rows_buf, osem):
            @pl.when(m >= 1)
            def _wait_prev():
                pltpu.make_async_copy(rows_buf, out_hbm.at[wid, 0],
                                      osem).wait()

            def group(g, carry):
                v = idx_v[pl.ds(j * C + g * 16, 16)]
                a = v * OUT_DIM
                base = g * (16 * OUT_DIM)
                for k in range(16):
                    addr = bcast_lane(a, k) + iota16
                    row = plsc.load_gather(table_v, [addr])
                    rows_buf[pl.ds(base + k * OUT_DIM, OUT_DIM)] = row
                return carry

            lax.fori_loop(0, groups, group, 0, unroll=2)
            pltpu.async_copy(rows_buf, out_hbm.at[wid, j], osem)

        def body(m, carry):
            do_chunk(2 * m, m, rows0, osem0)
            do_chunk(2 * m + 1, m, rows1, osem1)
            return carry

        lax.fori_loop(0, nchunks // 2, body, 0)
        pltpu.make_async_copy(rows0, out_hbm.at[wid, 0], osem0).wait()
        pltpu.make_async_copy(rows1, out_hbm.at[wid, 0], osem1).wait()

    return gather_kernel(table_flat, idx2d)


def kernel(x, embed, W1, b1, alpha, W2, b2):
    B = x.size
    info = plsc.get_sparse_core_info()
    nc, ns = info.num_cores, info.num_subcores
    nw = nc * ns
    C = 2560
    per_w = B // nw
    nchunks = per_w // C
    assert B == nw * nchunks * C and nchunks % 2 == 0, (B, nw, nchunks, C)

    table = _mlp_table(embed, W1, b1, alpha, W2, b2)
    idx2d = x.reshape(nw, per_w)
    out = _sc_gather(table.reshape(-1), idx2d, nc, ns, nchunks, C)
    return out.reshape(B, OUT_DIM)


# speedup vs baseline: 1.5351x; 1.2352x over previous
import functools

import jax
import jax.numpy as jnp
from jax import lax
from jax.experimental import pallas as pl
from jax.experimental.pallas import tpu as pltpu
from jax.experimental.pallas import tpu_sc as plsc

OUT_DIM = 16


def _mlp_table_body(embed_ref, W1_ref, b1_ref, alpha_ref, W2_ref, b2_ref,
                    out_ref):
    e = embed_ref[...]
    h = lax.dot(e, W1_ref[...], preferred_element_type=jnp.float32)
    h = h + b1_ref[...]
    a = alpha_ref[0, 0]
    h = jnp.maximum(h, 0.0) + a * jnp.minimum(h, 0.0)
    out_ref[...] = (lax.dot(h, W2_ref[...], preferred_element_type=jnp.float32)
                    + b2_ref[...])


def _mlp_table(embed, W1, b1, alpha, W2, b2):
    vocab = embed.shape[0]
    return pl.pallas_call(
        _mlp_table_body,
        out_shape=jax.ShapeDtypeStruct((vocab, OUT_DIM), jnp.float32),
        in_specs=[
            pl.BlockSpec(memory_space=pltpu.VMEM),
            pl.BlockSpec(memory_space=pltpu.VMEM),
            pl.BlockSpec(memory_space=pltpu.VMEM),
            pl.BlockSpec(memory_space=pltpu.SMEM),
            pl.BlockSpec(memory_space=pltpu.VMEM),
            pl.BlockSpec(memory_space=pltpu.VMEM),
        ],
        out_specs=pl.BlockSpec(memory_space=pltpu.VMEM),
    )(embed, W1, b1.reshape(1, -1), alpha.reshape(1, 1), W2,
      b2.reshape(1, -1))


def _sc_gather(table_flat, idx2d, nc, ns, nchunks, C):
    nw, per_w = idx2d.shape
    tlen = table_flat.shape[0]
    B = nw * per_w
    mesh = plsc.VectorSubcoreMesh(core_axis_name="c", subcore_axis_name="s")
    groups = C // 16

    @functools.partial(
        pl.kernel,
        out_type=jax.ShapeDtypeStruct((B, OUT_DIM), jnp.float32),
        mesh=mesh,
        scratch_types=[
            pltpu.VMEM((tlen,), jnp.float32),
            pltpu.VMEM((per_w,), jnp.int32),
            pltpu.VMEM((C, OUT_DIM), jnp.float32),
            pltpu.VMEM((C, OUT_DIM), jnp.float32),
            pltpu.SemaphoreType.DMA,
            pltpu.SemaphoreType.DMA,
        ],
        compiler_params=pltpu.CompilerParams(use_tc_tiling_on_sc=True,
                                             needs_layout_passes=False),
    )
    def gather_kernel(table_hbm, idx_hbm, out_hbm, table_v, idx_v,
                      rows0, rows1, osem0, osem1):
        wid = lax.axis_index("s") * nc + lax.axis_index("c")
        row0 = wid * per_w
        pltpu.sync_copy(table_hbm, table_v)
        pltpu.sync_copy(idx_hbm.at[wid], idx_v)
        iota16 = lax.iota(jnp.int32, 16)
        lane_sel = [jnp.full((16, 1), k, jnp.int32) for k in range(16)]
        dnums = lax.GatherDimensionNumbers(offset_dims=(),
                                           collapsed_slice_dims=(0,),
                                           start_index_map=(0,))

        def bcast_lane(vec, k):
            return lax.gather(vec, lane_sel[k], dnums, slice_sizes=(1,),
                              mode=lax.GatherScatterMode.PROMISE_IN_BOUNDS)

        def do_chunk(j, m, rows_buf, osem):
            @pl.when(m >= 1)
            def _wait_prev():
                pltpu.make_async_copy(rows_buf, out_hbm.at[pl.ds(0, C)],
                                      osem).wait()

            def group(g, carry):
                v = idx_v[pl.ds(j * C + g * 16, 16)]
                a = v * OUT_DIM
                for k in range(16):
                    addr = bcast_lane(a, k) + iota16
                    row = plsc.load_gather(table_v, [addr])
                    plsc.store_scatter(
                        rows_buf,
                        [jnp.full((16,), g * 16 + k, jnp.int32), iota16],
                        row)
                return carry

            lax.fori_loop(0, groups, group, 0, unroll=2)

            pltpu.async_copy(rows_buf, out_hbm.at[pl.ds(row0 + j * C, C)],
                             osem)

        def body(m, carry):
            do_chunk(2 * m, m, rows0, osem0)
            do_chunk(2 * m + 1, m, rows1, osem1)
            return carry

        lax.fori_loop(0, nchunks // 2, body, 0)
        pltpu.make_async_copy(rows0, out_hbm.at[pl.ds(0, C)], osem0).wait()
        pltpu.make_async_copy(rows1, out_hbm.at[pl.ds(0, C)], osem1).wait()

    return gather_kernel(table_flat, idx2d)


def kernel(x, embed, W1, b1, alpha, W2, b2):
    B = x.size
    info = plsc.get_sparse_core_info()
    nc, ns = info.num_cores, info.num_subcores
    nw = nc * ns
    C = 320
    S = 4
    per_w = B // (S * nw)
    nchunks = per_w // C
    assert B == S * nw * nchunks * C and nchunks % 2 == 0, (B, nw, nchunks)

    table = _mlp_table(embed, W1, b1, alpha, W2, b2)
    tflat = table.reshape(-1)
    xp = x.reshape(S, nw, per_w)
    parts = [_sc_gather(tflat, xp[s], nc, ns, nchunks, C) for s in range(S)]
    return jnp.concatenate(parts, axis=0)


# speedup vs baseline: 1.6348x; 1.0649x over previous
import functools

import jax
import jax.numpy as jnp
from jax import lax
from jax.experimental import pallas as pl
from jax.experimental.pallas import tpu as pltpu
from jax.experimental.pallas import tpu_sc as plsc

OUT_DIM = 16


def _mlp_table_body(embed_ref, W1_ref, b1_ref, alpha_ref, W2_ref, b2_ref,
                    out_ref):
    e = embed_ref[...]
    h = lax.dot(e, W1_ref[...], preferred_element_type=jnp.float32)
    h = h + b1_ref[...]
    a = alpha_ref[0, 0]
    h = jnp.maximum(h, 0.0) + a * jnp.minimum(h, 0.0)
    out_ref[...] = (lax.dot(h, W2_ref[...], preferred_element_type=jnp.float32)
                    + b2_ref[...])


def _mlp_table(embed, W1, b1, alpha, W2, b2):
    vocab = embed.shape[0]
    return pl.pallas_call(
        _mlp_table_body,
        out_shape=jax.ShapeDtypeStruct((vocab, OUT_DIM), jnp.float32),
        in_specs=[
            pl.BlockSpec(memory_space=pltpu.VMEM),
            pl.BlockSpec(memory_space=pltpu.VMEM),
            pl.BlockSpec(memory_space=pltpu.VMEM),
            pl.BlockSpec(memory_space=pltpu.SMEM),
            pl.BlockSpec(memory_space=pltpu.VMEM),
            pl.BlockSpec(memory_space=pltpu.VMEM),
        ],
        out_specs=pl.BlockSpec(memory_space=pltpu.VMEM),
    )(embed, W1, b1.reshape(1, -1), alpha.reshape(1, 1), W2,
      b2.reshape(1, -1))


def _sc_gather(table_flat, idx2d, nc, ns, nchunks, C):
    nw, per_w = idx2d.shape
    tlen = table_flat.shape[0]
    B = nw * per_w
    mesh = plsc.VectorSubcoreMesh(core_axis_name="c", subcore_axis_name="s")
    groups = C // 16

    @functools.partial(
        pl.kernel,
        out_type=jax.ShapeDtypeStruct((B, OUT_DIM), jnp.float32),
        mesh=mesh,
        scratch_types=[
            pltpu.VMEM((tlen,), jnp.float32),
            pltpu.VMEM((per_w,), jnp.int32),
            pltpu.VMEM((C, OUT_DIM), jnp.float32),
            pltpu.VMEM((C, OUT_DIM), jnp.float32),
            pltpu.SemaphoreType.DMA,
            pltpu.SemaphoreType.DMA,
        ],
        compiler_params=pltpu.CompilerParams(use_tc_tiling_on_sc=True,
                                             needs_layout_passes=False),
    )
    def gather_kernel(table_hbm, idx_hbm, out_hbm, table_v, idx_v,
                      rows0, rows1, osem0, osem1):
        wid = lax.axis_index("s") * nc + lax.axis_index("c")
        row0 = wid * per_w
        pltpu.sync_copy(table_hbm, table_v)
        pltpu.sync_copy(idx_hbm.at[wid], idx_v)
        iota16 = lax.iota(jnp.int32, 16)
        lane_sel = [jnp.full((16, 1), k, jnp.int32) for k in range(16)]
        dnums = lax.GatherDimensionNumbers(offset_dims=(),
                                           collapsed_slice_dims=(0,),
                                           start_index_map=(0,))

        def bcast_lane(vec, k):
            return lax.gather(vec, lane_sel[k], dnums, slice_sizes=(1,),
                              mode=lax.GatherScatterMode.PROMISE_IN_BOUNDS)

        def do_chunk(j, m, rows_buf, osem):
            @pl.when(m >= 1)
            def _wait_prev():
                pltpu.make_async_copy(rows_buf, out_hbm.at[pl.ds(0, C)],
                                      osem).wait()

            def group(g, carry):
                v = idx_v[pl.ds(j * C + g * 16, 16)]
                a = v * OUT_DIM
                for k in range(16):
                    addr = bcast_lane(a, k) + iota16
                    row = plsc.load_gather(table_v, [addr])
                    plsc.store_scatter(
                        rows_buf,
                        [jnp.full((16,), g * 16 + k, jnp.int32), iota16],
                        row)
                return carry

            lax.fori_loop(0, groups, group, 0, unroll=2)

            pltpu.async_copy(rows_buf, out_hbm.at[pl.ds(row0 + j * C, C)],
                             osem)

        def body(m, carry):
            do_chunk(2 * m, m, rows0, osem0)
            do_chunk(2 * m + 1, m, rows1, osem1)
            return carry

        lax.fori_loop(0, nchunks // 2, body, 0)
        pltpu.make_async_copy(rows0, out_hbm.at[pl.ds(0, C)], osem0).wait()
        pltpu.make_async_copy(rows1, out_hbm.at[pl.ds(0, C)], osem1).wait()

    return gather_kernel(table_flat, idx2d)


def kernel(x, embed, W1, b1, alpha, W2, b2):
    B = x.size
    info = plsc.get_sparse_core_info()
    nc, ns = info.num_cores, info.num_subcores
    nw = nc * ns
    C = 320
    per_w = B // nw
    nchunks = per_w // C
    assert B == nw * nchunks * C and nchunks % 2 == 0, (B, nw, nchunks, C)

    table = _mlp_table(embed, W1, b1, alpha, W2, b2)
    idx2d = x.reshape(nw, per_w)
    return _sc_gather(table.reshape(-1), idx2d, nc, ns, nchunks, C)
